# baseline (speedup 1.0000x reference)
import functools

import jax
import jax.numpy as jnp
from jax import lax
from jax.experimental import pallas as pl
from jax.experimental.pallas import tpu as pltpu

B, C, H, W = 4, 192, 384, 384
E = 16
CH = C // 4
ROWS = B * C
COLS = H * W
SEG = 4096
NSEG = COLS // SEG
SROWS = ROWS * NSEG
BR = 512
NSTEP = SROWS // BR


def _reduce_body(x_ref, o_ref):
    blk = x_ref[...]
    acc = blk[:, 0:128]
    for j in range(1, SEG // 128):
        acc = acc + blk[:, j * 128:(j + 1) * 128]
    o_ref[...] = acc


def _mlp_body(p_ref, w1_ref, b1_ref, w2_ref, b2_ref, o_ref):
    p = p_ref[...]
    pooled = p.sum(axis=1) * (1.0 / COLS)
    pooled = pooled.reshape(B, C)
    h = lax.dot_general(pooled, w1_ref[...],
                        (((1,), (1,)), ((), ())),
                        preferred_element_type=jnp.float32)
    h = jnp.maximum(h + b1_ref[...], 0.0)
    logits = lax.dot_general(h, w2_ref[...],
                             (((1,), (1,)), ((), ())),
                             preferred_element_type=jnp.float32)
    logits = logits + b2_ref[...]
    m = jnp.max(logits, axis=1, keepdims=True)
    e = jnp.exp(logits - m)
    o_ref[...] = e / jnp.sum(e, axis=1, keepdims=True)


@jax.jit
def kernel(x, w1, b1, w2, b2):
    x2 = x.reshape(SROWS, SEG)
    partials = pl.pallas_call(
        _reduce_body,
        grid=(NSTEP,),
        in_specs=[pl.BlockSpec((BR, SEG), lambda i: (i, 0))],
        out_specs=pl.BlockSpec((BR, 128), lambda i: (i, 0)),
        out_shape=jax.ShapeDtypeStruct((SROWS, 128), jnp.float32),
    )(x2)
    out = pl.pallas_call(
        _mlp_body,
        out_shape=jax.ShapeDtypeStruct((B, E), jnp.float32),
    )(partials.reshape(ROWS, NSEG * 128), w1, b1.reshape(1, CH),
      w2, b2.reshape(1, E))
    return out


# speedup vs baseline: 4.2563x; 4.2563x over previous
import functools

import jax
import jax.numpy as jnp
from jax import lax
from jax.experimental import pallas as pl
from jax.experimental.pallas import tpu as pltpu

B, C, H, W = 4, 192, 384, 384
E = 16
CH = C // 4
ROWS = B * C
COLS = H * W
CB = 16
NH = H // 8
NW = W // 128


def _reduce_body(x_ref, o_ref):
    x4 = x_ref[...]
    acc = x4[0, :, 0:8, 0:128]
    for hg in range(NH):
        for wg in range(NW):
            if hg == 0 and wg == 0:
                continue
            acc = acc + x4[0, :, hg * 8:hg * 8 + 8, wg * 128:wg * 128 + 128]
    o_ref[...] = acc[None]


def _mlp_body(p_ref, w1_ref, b1_ref, w2_ref, b2_ref, o_ref):
    p = p_ref[...]
    pooled = p.sum(axis=(2, 3)) * (1.0 / COLS)
    h = lax.dot_general(pooled, w1_ref[...],
                        (((1,), (1,)), ((), ())),
                        preferred_element_type=jnp.float32)
    h = jnp.maximum(h + b1_ref[...], 0.0)
    logits = lax.dot_general(h, w2_ref[...],
                             (((1,), (1,)), ((), ())),
                             preferred_element_type=jnp.float32)
    logits = logits + b2_ref[...]
    m = jnp.max(logits, axis=1, keepdims=True)
    e = jnp.exp(logits - m)
    o_ref[...] = e / jnp.sum(e, axis=1, keepdims=True)


@jax.jit
def kernel(x, w1, b1, w2, b2):
    partials = pl.pallas_call(
        _reduce_body,
        grid=(B, C // CB),
        in_specs=[pl.BlockSpec((1, CB, H, W), lambda b, c: (b, c, 0, 0))],
        out_specs=pl.BlockSpec((1, CB, 8, 128), lambda b, c: (b, c, 0, 0)),
        out_shape=jax.ShapeDtypeStruct((B, C, 8, 128), jnp.float32),
    )(x)
    out = pl.pallas_call(
        _mlp_body,
        out_shape=jax.ShapeDtypeStruct((B, E), jnp.float32),
    )(partials, w1, b1.reshape(1, CH), w2, b2.reshape(1, E))
    return out


# speedup vs baseline: 4.3617x; 1.0248x over previous
import functools

import jax
import jax.numpy as jnp
from jax import lax
from jax.experimental import pallas as pl
from jax.experimental.pallas import tpu as pltpu

B, C, H, W = 4, 192, 384, 384
E = 16
CH = C // 4
COLS = H * W
CB = 24
NC = C // CB
NH = H // 8
NW = W // 128


def _body(x_ref, w1_ref, b1_ref, w2_ref, b2_ref, o_ref, pacc_ref):
    x4 = x_ref[...]
    acc = x4[0, :, 0:8, 0:128]
    for hg in range(NH):
        for wg in range(NW):
            if hg == 0 and wg == 0:
                continue
            acc = acc + x4[0, :, hg * 8:hg * 8 + 8, wg * 128:wg * 128 + 128]
    b = pl.program_id(0)
    cb = pl.program_id(1)
    pacc_ref[b, pl.ds(cb * CB, CB), :] = acc.sum(axis=1)

    @pl.when((b == B - 1) & (cb == NC - 1))
    def _():
        pooled = pacc_ref[...].sum(axis=2) * (1.0 / COLS)
        h = lax.dot_general(pooled, w1_ref[...],
                            (((1,), (1,)), ((), ())),
                            preferred_element_type=jnp.float32)
        h = jnp.maximum(h + b1_ref[...], 0.0)
        logits = lax.dot_general(h, w2_ref[...],
                                 (((1,), (1,)), ((), ())),
                                 preferred_element_type=jnp.float32)
        logits = logits + b2_ref[...]
        m = jnp.max(logits, axis=1, keepdims=True)
        e = jnp.exp(logits - m)
        o_ref[...] = e / jnp.sum(e, axis=1, keepdims=True)


@jax.jit
def kernel(x, w1, b1, w2, b2):
    return pl.pallas_call(
        _body,
        grid=(B, NC),
        in_specs=[
            pl.BlockSpec((1, CB, H, W), lambda b, c: (b, c, 0, 0)),
            pl.BlockSpec((CH, C), lambda b, c: (0, 0)),
            pl.BlockSpec((1, CH), lambda b, c: (0, 0)),
            pl.BlockSpec((E, CH), lambda b, c: (0, 0)),
            pl.BlockSpec((1, E), lambda b, c: (0, 0)),
        ],
        out_specs=pl.BlockSpec((B, E), lambda b, c: (0, 0)),
        out_shape=jax.ShapeDtypeStruct((B, E), jnp.float32),
        scratch_shapes=[pltpu.VMEM((B, C, 128), jnp.float32)],
    )(x, w1, b1.reshape(1, CH), w2, b2.reshape(1, E))
